# baseline (speedup 1.0000x reference)
import functools

import jax
import jax.numpy as jnp
from jax import lax
from jax.experimental import pallas as pl
from jax.experimental.pallas import tpu as pltpu
from jax.experimental.pallas import tpu_sc as plsc

_f32 = jnp.float32


def _relu(v):
    return jnp.maximum(v, 0.0)


def _dot(a, b):
    return jnp.dot(a.astype(jnp.bfloat16), b.astype(jnp.bfloat16),
                   preferred_element_type=_f32)


def _proj1_body(x_ref, wd_ref, ws_ref, bd_ref, o_ref):
    x = x_ref[...]
    hd = _dot(x, wd_ref[...]) + bd_ref[...]
    hs = _dot(x, ws_ref[...])
    pad = jnp.zeros((x.shape[0], 128 - hd.shape[1]), _f32)
    o_ref[0] = jnp.concatenate([hd, pad], axis=1)
    o_ref[1] = jnp.concatenate([hs, pad], axis=1)


def _proj1_call(x, wd, ws, bd):
    n = x.shape[0]
    return pl.pallas_call(
        _proj1_body,
        out_shape=jax.ShapeDtypeStruct((2, n, 128), _f32),
    )(x, wd, ws, bd)


def _edge1_body(gd_ref, gs_ref, ea_ref, we, w2, b2, w3, b3, w4, b4, o_ref):
    h = we.shape[1]
    t = _relu(gd_ref[:, :h] + gs_ref[:, :h] + _dot(ea_ref[...], we[...]))
    t = _relu(_dot(t, w2[...]) + b2[...])
    t = _relu(_dot(t, w3[...]) + b3[...])
    o_ref[...] = _dot(t, w4[...]) + b4[...]


def _edge1_call(g, ea, we, w2, b2, w3, b3, w4, b4, blk, e_pad):
    e = ea.shape[0]
    nb = e // blk
    full = lambda a: pl.BlockSpec(a.shape, lambda i: (0,) * a.ndim)
    de = w4.shape[1]
    return pl.pallas_call(
        _edge1_body,
        grid=(nb,),
        in_specs=[
            pl.BlockSpec((blk, 128), lambda i: (i, 0)),
            pl.BlockSpec((blk, 128), lambda i, _nb=nb: (i + _nb, 0)),
            pl.BlockSpec((blk, ea.shape[1]), lambda i: (i, 0)),
            full(we), full(w2), full(b2), full(w3), full(b3),
            full(w4), full(b4),
        ],
        out_specs=pl.BlockSpec((blk, de), lambda i: (i, 0)),
        out_shape=jax.ShapeDtypeStruct((e_pad, de), _f32),
    )(g, g, ea, we, w2, b2, w3, b3, w4, b4)


def _node2_body(x_ref, agg_ref, wxa, wxb, bx1, w2, b2, w3, b3, w4, b4,
                wd, ws, bd, o_ref):
    x = x_ref[...]
    agg = agg_ref[...]
    t = _relu(_dot(x, wxa[...]) + _dot(agg, wxb[...]) + bx1[...])
    t = _relu(_dot(t, w2[...]) + b2[...])
    t = _relu(_dot(t, w3[...]) + b3[...])
    x1 = _dot(t, w4[...]) + b4[...]
    hd = _dot(x1, wd[...]) + bd[...]
    hs = _dot(x1, ws[...])
    pad = jnp.zeros((x.shape[0], 128 - hd.shape[1]), _f32)
    o_ref[0] = jnp.concatenate([hd, pad], axis=1)
    o_ref[1] = jnp.concatenate([hs, pad], axis=1)


def _reduce_body(p_ref, o_ref):
    @pl.when(pl.program_id(1) == 0)
    def _():
        o_ref[...] = jnp.zeros_like(o_ref)

    o_ref[...] += p_ref[0, 0]


def _reduce_call(p, n_nodes):
    _, _, half, d = p.shape
    return pl.pallas_call(
        _reduce_body,
        grid=(2, 16),
        in_specs=[pl.BlockSpec((1, 1, half, d), lambda h, t: (h, t, 0, 0))],
        out_specs=pl.BlockSpec((half, d), lambda h, t: (h, 0)),
        out_shape=jax.ShapeDtypeStruct((n_nodes, d), _f32),
    )(p)


def _reduce_body(p_ref, o_ref):
    @pl.when(pl.program_id(1) == 0)
    def _():
        o_ref[...] = jnp.zeros_like(o_ref)

    o_ref[...] += p_ref[0, 0]


def _reduce_call(p, n_nodes):
    _, _, half, d = p.shape
    return pl.pallas_call(
        _reduce_body,
        grid=(2, 16),
        in_specs=[pl.BlockSpec((1, 1, half, d), lambda h, t: (h, t, 0, 0))],
        out_specs=pl.BlockSpec((half, d), lambda h, t: (h, 0)),
        out_shape=jax.ShapeDtypeStruct((n_nodes, d), _f32),
    )(p)


def _node2_call(x, agg, wxa, wxb, bx1, w2, b2, w3, b3, w4, b4, wd, ws, bd):
    n = x.shape[0]
    return pl.pallas_call(
        _node2_body,
        out_shape=jax.ShapeDtypeStruct((2, n, 128), _f32),
    )(x, agg, wxa, wxb, bx1, w2, b2, w3, b3, w4, b4, wd, ws, bd)


def _edge2_body(gd_ref, gs_ref, e1_ref, ea_ref, we, w2, b2, w3, b3, w4, b4,
                va, vb, vc, c1, v2, c2, v3, c3, v4, c4, o_ref):
    h = we.shape[1]
    e1 = e1_ref[...]
    ea = ea_ref[...]
    t = _relu(gd_ref[:, :h] + gs_ref[:, :h] + _dot(e1, we[...]))
    t = _relu(_dot(t, w2[...]) + b2[...])
    t = _relu(_dot(t, w3[...]) + b3[...])
    e2 = _dot(t, w4[...]) + b4[...]
    z = _relu(_dot(ea, va[...]) + _dot(e1, vb[...]) + _dot(e2, vc[...])
              + c1[...])
    z = _relu(_dot(z, v2[...]) + c2[...])
    z = _relu(_dot(z, v3[...]) + c3[...])
    logits = _dot(z, v4[...]) + c4[...]
    o_ref[...] = 1.0 / (1.0 + jnp.exp(-logits))


def _edge2_call(g, e1, ea, weights, blk):
    e = ea.shape[0]
    nb = e // blk
    de = ea.shape[1]
    full = lambda a: pl.BlockSpec(a.shape, lambda i: (0,) * a.ndim)
    return pl.pallas_call(
        _edge2_body,
        grid=(nb,),
        in_specs=[
            pl.BlockSpec((blk, 128), lambda i: (i, 0)),
            pl.BlockSpec((blk, 128), lambda i, _nb=nb: (i + _nb, 0)),
            pl.BlockSpec((blk, de), lambda i: (i, 0)),
            pl.BlockSpec((blk, de), lambda i: (i, 0)),
        ] + [full(w) for w in weights],
        out_specs=pl.BlockSpec((blk, 1), lambda i: (i, 0)),
        out_shape=jax.ShapeDtypeStruct((e, 1), _f32),
    )(g, g, e1, ea, *weights)


_GW = 128


def _sc_gather(table, idx):
    m = idx.shape[0]
    d = table.shape[1]
    mesh = plsc.VectorSubcoreMesh(core_axis_name="c", subcore_axis_name="s")

    @functools.partial(
        pl.kernel,
        out_type=jax.ShapeDtypeStruct((m, d), _f32),
        mesh=mesh,
    )
    def k(tab_hbm, i_hbm, o_hbm):
        def body(i_vmem, o_vmem):
            pltpu.sync_copy(tab_hbm.at[i_vmem.at[0]], o_vmem)

        pltpu.emit_pipeline(
            body,
            grid=(m // _GW,),
            in_specs=[pl.BlockSpec((1, _GW), lambda i: (0, i))],
            out_specs=[pl.BlockSpec((_GW, d), lambda i: (i, 0))],
            core_axis_name=("c", "s"),
            dimension_semantics=(pltpu.PARALLEL,),
        )(i_hbm, o_hbm)

    return k(table, idx.reshape(1, m))


def _sc_scatter_add(e_pad, dst2d, n_nodes):
    ne, d = e_pad.shape
    half = n_nodes // 2
    n_rows = ne // 128
    rows_per_tile = n_rows // 32
    n_ch = rows_per_tile // 4
    mesh = plsc.VectorSubcoreMesh(core_axis_name="c", subcore_axis_name="s")

    @functools.partial(
        pl.kernel,
        out_type=jax.ShapeDtypeStruct((2, 16, half, d), _f32),
        mesh=mesh,
        scratch_types=[
            pltpu.VMEM((512, d), _f32),
            pltpu.VMEM((4, 128), jnp.int32),
            pltpu.VMEM((4, 128), jnp.int32),
            pltpu.VMEM_SHARED((half + 8, d), _f32),
        ],
    )
    def k(e_hbm, d_hbm, z_hbm, o_hbm, ebuf, ibuf, lbuf, acc):
        cid = lax.axis_index("c")
        sid = lax.axis_index("s")
        wid = cid * 16 + sid
        grp = sid // 8
        pos = cid * 8 + sid % 8
        lo = grp * half

        pltpu.sync_copy(z_hbm, acc)

        base = wid * rows_per_tile

        @pl.loop(0, n_ch)
        def _(c):
            row0 = base + c * 4
            pltpu.sync_copy(e_hbm.at[pl.ds(row0 * 128, 512)], ebuf)
            pltpu.sync_copy(d_hbm.at[pl.ds(row0, 4)], ibuf)

            for r in range(4):
                for v in range(8):
                    idx = ibuf[r, pl.ds(v * 16, 16)]
                    loc = idx - lo
                    ok = (loc >= 0) & (loc < half)
                    lbuf[r, pl.ds(v * 16, 16)] = jnp.where(
                        ok, loc, jnp.full((16,), half, jnp.int32)
                    )

            for r in range(4):
                pltpu.sync_copy(
                    ebuf.at[pl.ds(r * 128, 128)],
                    acc.at[lbuf.at[r]],
                    add=True,
                )

        pltpu.sync_copy(acc.at[pl.ds(0, half)], o_hbm.at[grp, pos])

    z = jnp.zeros((half + 8, d), _f32)
    return k(e_pad, dst2d, z)


_EDGE_BLK = 2000


def kernel(x, edge_index, edge_attr, params):
    n, dn = x.shape
    e_cnt = edge_index.shape[1]
    de = edge_attr.shape[1]
    r1, o1, r2, w = params["r1"], params["o1"], params["r2"], params["w"]

    src = edge_index[0]
    dst = edge_index[1]
    idxg = jnp.concatenate([dst, src + n])
    e_pad = 32 * 1024 * -(-e_cnt // (32 * 1024))
    dst2d = jnp.concatenate(
        [dst, jnp.full((e_pad - e_cnt,), n, jnp.int32)]
    ).reshape(-1, 128)

    row = lambda v: v.reshape(1, -1)

    w1, b1 = r1[0]
    t1 = _proj1_call(x, w1[:dn], w1[dn:2 * dn], row(b1))
    g1 = _sc_gather(t1.reshape(2 * n, 128), idxg)
    e1p = _edge1_call(
        g1, edge_attr, w1[2 * dn:],
        r1[1][0], row(r1[1][1]), r1[2][0], row(r1[2][1]),
        r1[3][0], row(r1[3][1]), _EDGE_BLK, e_pad,
    )
    agg = _reduce_call(_sc_scatter_add(e1p, dst2d, n), n)

    wo1, bo1 = o1[0]
    w21, b21 = r2[0]
    t2 = _node2_call(
        x, agg, wo1[:dn], wo1[dn:], row(bo1),
        o1[1][0], row(o1[1][1]), o1[2][0], row(o1[2][1]),
        o1[3][0], row(o1[3][1]),
        w21[:dn], w21[dn:2 * dn], row(b21),
    )

    g2 = _sc_gather(t2.reshape(2 * n, 128), idxg)
    wv1, cv1 = w[0]
    weights = [
        w21[2 * dn:],
        r2[1][0], row(r2[1][1]), r2[2][0], row(r2[2][1]),
        r2[3][0], row(r2[3][1]),
        wv1[:de], wv1[de:2 * de], wv1[2 * de:], row(cv1),
        w[1][0], row(w[1][1]), w[2][0], row(w[2][1]),
        w[3][0], row(w[3][1]),
    ]
    return _edge2_call(g2, e1p, edge_attr, weights, _EDGE_BLK)


# speedup vs baseline: 1.2263x; 1.2263x over previous
import functools

import jax
import jax.numpy as jnp
from jax import lax
from jax.experimental import pallas as pl
from jax.experimental.pallas import tpu as pltpu
from jax.experimental.pallas import tpu_sc as plsc

_f32 = jnp.float32


def _relu(v):
    return jnp.maximum(v, 0.0)


def _dot(a, b):
    return jnp.dot(a, b, preferred_element_type=_f32)


def _proj1_body(x_ref, wd_ref, ws_ref, bd_ref, o_ref):
    x = x_ref[...]
    hd = _dot(x, wd_ref[...]) + bd_ref[...]
    hs = _dot(x, ws_ref[...])
    pad = jnp.zeros((x.shape[0], 128 - hd.shape[1]), _f32)
    o_ref[0] = jnp.concatenate([hd, pad], axis=1)
    o_ref[1] = jnp.concatenate([hs, pad], axis=1)


def _proj1_call(x, wd, ws, bd):
    n = x.shape[0]
    return pl.pallas_call(
        _proj1_body,
        out_shape=jax.ShapeDtypeStruct((2, n, 128), _f32),
    )(x, wd, ws, bd)


def _edge1_body(gd_ref, gs_ref, ea_ref, we, w2, b2, w3, b3, w4, b4, o_ref):
    h = we.shape[1]
    t = _relu(gd_ref[:, :h] + gs_ref[:, :h] + _dot(ea_ref[...], we[...]))
    t = _relu(_dot(t, w2[...]) + b2[...])
    t = _relu(_dot(t, w3[...]) + b3[...])
    o_ref[...] = _dot(t, w4[...]) + b4[...]


def _edge1_call(g, ea, we, w2, b2, w3, b3, w4, b4, blk, e_pad):
    e = ea.shape[0]
    nb = e // blk
    full = lambda a: pl.BlockSpec(a.shape, lambda i: (0,) * a.ndim)
    de = w4.shape[1]
    return pl.pallas_call(
        _edge1_body,
        grid=(nb,),
        in_specs=[
            pl.BlockSpec((blk, 128), lambda i: (i, 0)),
            pl.BlockSpec((blk, 128), lambda i, _nb=nb: (i + _nb, 0)),
            pl.BlockSpec((blk, ea.shape[1]), lambda i: (i, 0)),
            full(we), full(w2), full(b2), full(w3), full(b3),
            full(w4), full(b4),
        ],
        out_specs=pl.BlockSpec((blk, de), lambda i: (i, 0)),
        out_shape=jax.ShapeDtypeStruct((e_pad, de), _f32),
    )(g, g, ea, we, w2, b2, w3, b3, w4, b4)


def _node2_body(x_ref, agg_ref, wxa, wxb, bx1, w2, b2, w3, b3, w4, b4,
                wd, ws, bd, o_ref):
    x = x_ref[...]
    agg = agg_ref[...]
    t = _relu(_dot(x, wxa[...]) + _dot(agg, wxb[...]) + bx1[...])
    t = _relu(_dot(t, w2[...]) + b2[...])
    t = _relu(_dot(t, w3[...]) + b3[...])
    x1 = _dot(t, w4[...]) + b4[...]
    hd = _dot(x1, wd[...]) + bd[...]
    hs = _dot(x1, ws[...])
    pad = jnp.zeros((x.shape[0], 128 - hd.shape[1]), _f32)
    o_ref[0] = jnp.concatenate([hd, pad], axis=1)
    o_ref[1] = jnp.concatenate([hs, pad], axis=1)


def _reduce_body(p_ref, o_ref):
    @pl.when(pl.program_id(1) == 0)
    def _():
        o_ref[...] = jnp.zeros_like(o_ref)

    o_ref[...] += p_ref[0, 0]


def _reduce_call(p, n_nodes):
    _, _, half, d = p.shape
    return pl.pallas_call(
        _reduce_body,
        grid=(2, 16),
        in_specs=[pl.BlockSpec((1, 1, half, d), lambda h, t: (h, t, 0, 0))],
        out_specs=pl.BlockSpec((half, d), lambda h, t: (h, 0)),
        out_shape=jax.ShapeDtypeStruct((n_nodes, d), _f32),
    )(p)


def _reduce_body(p_ref, o_ref):
    @pl.when(pl.program_id(1) == 0)
    def _():
        o_ref[...] = jnp.zeros_like(o_ref)

    o_ref[...] += p_ref[0, 0]


def _reduce_call(p, n_nodes):
    _, _, half, d = p.shape
    return pl.pallas_call(
        _reduce_body,
        grid=(2, 16),
        in_specs=[pl.BlockSpec((1, 1, half, d), lambda h, t: (h, t, 0, 0))],
        out_specs=pl.BlockSpec((half, d), lambda h, t: (h, 0)),
        out_shape=jax.ShapeDtypeStruct((n_nodes, d), _f32),
    )(p)


def _node2_call(x, agg, wxa, wxb, bx1, w2, b2, w3, b3, w4, b4, wd, ws, bd):
    n = x.shape[0]
    return pl.pallas_call(
        _node2_body,
        out_shape=jax.ShapeDtypeStruct((2, n, 128), _f32),
    )(x, agg, wxa, wxb, bx1, w2, b2, w3, b3, w4, b4, wd, ws, bd)


def _edge2_body(gd_ref, gs_ref, e1_ref, ea_ref, we, w2, b2, w3, b3, w4, b4,
                v1, c1, v2, c2, v3, c3, v4, c4, o_ref):
    h = we.shape[1]
    e1 = e1_ref[...]
    ea = ea_ref[...]
    t = _relu(gd_ref[:, :h] + gs_ref[:, :h] + _dot(e1, we[...]))
    t = _relu(_dot(t, w2[...]) + b2[...])
    t = _relu(_dot(t, w3[...]) + b3[...])
    e2 = _dot(t, w4[...]) + b4[...]
    z0 = jnp.concatenate([ea, e1, e2], axis=1)
    z = _relu(_dot(z0, v1[...]) + c1[...])
    z = _relu(_dot(z, v2[...]) + c2[...])
    z = _relu(_dot(z, v3[...]) + c3[...])
    logits = jnp.sum(z * v4[...], axis=1, keepdims=True) + c4[...]
    o_ref[...] = 1.0 / (1.0 + jnp.exp(-logits))


def _edge2_call(g, e1, ea, weights, blk):
    e = ea.shape[0]
    nb = e // blk
    de = ea.shape[1]
    full = lambda a: pl.BlockSpec(a.shape, lambda i: (0,) * a.ndim)
    return pl.pallas_call(
        _edge2_body,
        grid=(nb,),
        in_specs=[
            pl.BlockSpec((blk, 128), lambda i: (i, 0)),
            pl.BlockSpec((blk, 128), lambda i, _nb=nb: (i + _nb, 0)),
            pl.BlockSpec((blk, de), lambda i: (i, 0)),
            pl.BlockSpec((blk, de), lambda i: (i, 0)),
        ] + [full(w) for w in weights],
        out_specs=pl.BlockSpec((blk, 1), lambda i: (i, 0)),
        out_shape=jax.ShapeDtypeStruct((e, 1), _f32),
    )(g, g, e1, ea, *weights)


_GW = 128


def _sc_gather(table, idx):
    m = idx.shape[0]
    d = table.shape[1]
    mesh = plsc.VectorSubcoreMesh(core_axis_name="c", subcore_axis_name="s")

    @functools.partial(
        pl.kernel,
        out_type=jax.ShapeDtypeStruct((m, d), _f32),
        mesh=mesh,
    )
    def k(tab_hbm, i_hbm, o_hbm):
        def body(i_vmem, o_vmem):
            pltpu.sync_copy(tab_hbm.at[i_vmem.at[0]], o_vmem)

        pltpu.emit_pipeline(
            body,
            grid=(m // _GW,),
            in_specs=[pl.BlockSpec((1, _GW), lambda i: (0, i))],
            out_specs=[pl.BlockSpec((_GW, d), lambda i: (i, 0))],
            core_axis_name=("c", "s"),
            dimension_semantics=(pltpu.PARALLEL,),
        )(i_hbm, o_hbm)

    return k(table, idx.reshape(1, m))


def _sc_scatter_add(e_pad, dst2d, n_nodes):
    ne, d = e_pad.shape
    half = n_nodes // 2
    n_rows = ne // 128
    rows_per_tile = n_rows // 32
    n_ch = rows_per_tile // 4
    mesh = plsc.VectorSubcoreMesh(core_axis_name="c", subcore_axis_name="s")

    @functools.partial(
        pl.kernel,
        out_type=jax.ShapeDtypeStruct((2, 16, half, d), _f32),
        mesh=mesh,
        scratch_types=[
            pltpu.VMEM((512, d), _f32),
            pltpu.VMEM((4, 128), jnp.int32),
            pltpu.VMEM((4, 128), jnp.int32),
            pltpu.VMEM_SHARED((half + 8, d), _f32),
        ],
    )
    def k(e_hbm, d_hbm, z_hbm, o_hbm, ebuf, ibuf, lbuf, acc):
        cid = lax.axis_index("c")
        sid = lax.axis_index("s")
        wid = cid * 16 + sid
        grp = sid // 8
        pos = cid * 8 + sid % 8
        lo = grp * half

        pltpu.sync_copy(z_hbm, acc)

        base = wid * rows_per_tile

        @pl.loop(0, n_ch)
        def _(c):
            row0 = base + c * 4
            pltpu.sync_copy(e_hbm.at[pl.ds(row0 * 128, 512)], ebuf)
            pltpu.sync_copy(d_hbm.at[pl.ds(row0, 4)], ibuf)

            for r in range(4):
                for v in range(8):
                    idx = ibuf[r, pl.ds(v * 16, 16)]
                    loc = idx - lo
                    ok = (loc >= 0) & (loc < half)
                    lbuf[r, pl.ds(v * 16, 16)] = jnp.where(
                        ok, loc, jnp.full((16,), half, jnp.int32)
                    )

            for r in range(4):
                pltpu.sync_copy(
                    ebuf.at[pl.ds(r * 128, 128)],
                    acc.at[lbuf.at[r]],
                    add=True,
                )

        pltpu.sync_copy(acc.at[pl.ds(0, half)], o_hbm.at[grp, pos])

    z = jnp.zeros((half + 8, d), _f32)
    return k(e_pad, dst2d, z)


_EDGE_BLK = 8000


def kernel(x, edge_index, edge_attr, params):
    n, dn = x.shape
    e_cnt = edge_index.shape[1]
    de = edge_attr.shape[1]
    r1, o1, r2, w = params["r1"], params["o1"], params["r2"], params["w"]

    src = edge_index[0]
    dst = edge_index[1]
    idxg = jnp.concatenate([dst, src + n])
    e_pad = 32 * 1024 * -(-e_cnt // (32 * 1024))
    dst2d = jnp.concatenate(
        [dst, jnp.full((e_pad - e_cnt,), n, jnp.int32)]
    ).reshape(-1, 128)

    row = lambda v: v.reshape(1, -1)

    w1, b1 = r1[0]
    t1 = _proj1_call(x, w1[:dn], w1[dn:2 * dn], row(b1))
    g1 = _sc_gather(t1.reshape(2 * n, 128), idxg)
    e1p = _edge1_call(
        g1, edge_attr, w1[2 * dn:],
        r1[1][0], row(r1[1][1]), r1[2][0], row(r1[2][1]),
        r1[3][0], row(r1[3][1]), _EDGE_BLK, e_pad,
    )
    agg = _reduce_call(_sc_scatter_add(e1p, dst2d, n), n)

    wo1, bo1 = o1[0]
    w21, b21 = r2[0]
    t2 = _node2_call(
        x, agg, wo1[:dn], wo1[dn:], row(bo1),
        o1[1][0], row(o1[1][1]), o1[2][0], row(o1[2][1]),
        o1[3][0], row(o1[3][1]),
        w21[:dn], w21[dn:2 * dn], row(b21),
    )

    g2 = _sc_gather(t2.reshape(2 * n, 128), idxg)
    wv1, cv1 = w[0]
    weights = [
        w21[2 * dn:],
        r2[1][0], row(r2[1][1]), r2[2][0], row(r2[2][1]),
        r2[3][0], row(r2[3][1]),
        wv1, row(cv1),
        w[1][0], row(w[1][1]), w[2][0], row(w[2][1]),
        row(w[3][0].reshape(-1)), row(w[3][1]),
    ]
    return _edge2_call(g2, e1p, edge_attr, weights, _EDGE_BLK)


# speedup vs baseline: 1.2591x; 1.0268x over previous
import functools

import jax
import jax.numpy as jnp
from jax import lax
from jax.experimental import pallas as pl
from jax.experimental.pallas import tpu as pltpu
from jax.experimental.pallas import tpu_sc as plsc

_f32 = jnp.float32


def _relu(v):
    return jnp.maximum(v, 0.0)


def _dot(a, b):
    return jnp.dot(a, b, preferred_element_type=_f32)


def _proj1_body(x_ref, wd_ref, ws_ref, bd_ref, o_ref):
    x = x_ref[...]
    hd = _dot(x, wd_ref[...]) + bd_ref[...]
    hs = _dot(x, ws_ref[...])
    pad = jnp.zeros((x.shape[0], 128 - hd.shape[1]), _f32)
    o_ref[0] = jnp.concatenate([hd, pad], axis=1)
    o_ref[1] = jnp.concatenate([hs, pad], axis=1)


def _proj1_call(x, wd, ws, bd):
    n = x.shape[0]
    return pl.pallas_call(
        _proj1_body,
        out_shape=jax.ShapeDtypeStruct((2, n, 128), _f32),
    )(x, wd, ws, bd)


def _edge1_body(gd_ref, gs_ref, ea_ref, we, w2, b2, w3, b3, w4, b4, o_ref):
    h = we.shape[1]
    t = _relu(gd_ref[:, :h] + gs_ref[:, :h] + _dot(ea_ref[...], we[...]))
    t = _relu(_dot(t, w2[...]) + b2[...])
    t = _relu(_dot(t, w3[...]) + b3[...])
    o_ref[...] = _dot(t, w4[...]) + b4[...]


def _edge1_call(g, ea, ws, blk, eh, e_pad_h, ea_off):
    nb = eh // blk
    full = lambda a: pl.BlockSpec(a.shape, lambda i: (0,) * a.ndim)
    de = ea.shape[1]
    return pl.pallas_call(
        _edge1_body,
        grid=(nb,),
        in_specs=[
            pl.BlockSpec((blk, 128), lambda i: (i, 0)),
            pl.BlockSpec((blk, 128), lambda i: (i + nb, 0)),
            pl.BlockSpec((blk, de), lambda i: (i + ea_off, 0)),
        ] + [full(w) for w in ws],
        out_specs=pl.BlockSpec((blk, de), lambda i: (i, 0)),
        out_shape=jax.ShapeDtypeStruct((e_pad_h, de), _f32),
    )(g, g, ea, *ws)


def _reduce_body(p_ref, o_ref):
    @pl.when(pl.program_id(1) == 0)
    def _():
        o_ref[...] = jnp.zeros_like(o_ref)

    o_ref[...] += p_ref[0, 0]


def _reduce_call(p, n_nodes):
    _, _, half, d = p.shape
    return pl.pallas_call(
        _reduce_body,
        grid=(2, 16),
        in_specs=[pl.BlockSpec((1, 1, half, d), lambda h, t: (h, t, 0, 0))],
        out_specs=pl.BlockSpec((half, d), lambda h, t: (h, 0)),
        out_shape=jax.ShapeDtypeStruct((n_nodes, d), _f32),
    )(p)


def _node2_body(x_ref, agg_ref, wxa, wxb, bx1, w2, b2, w3, b3, w4, b4,
                wd, ws, bd, o_ref):
    x = x_ref[...]
    agg = agg_ref[...]
    t = _relu(_dot(x, wxa[...]) + _dot(agg, wxb[...]) + bx1[...])
    t = _relu(_dot(t, w2[...]) + b2[...])
    t = _relu(_dot(t, w3[...]) + b3[...])
    x1 = _dot(t, w4[...]) + b4[...]
    hd = _dot(x1, wd[...]) + bd[...]
    hs = _dot(x1, ws[...])
    pad = jnp.zeros((x.shape[0], 128 - hd.shape[1]), _f32)
    o_ref[0] = jnp.concatenate([hd, pad], axis=1)
    o_ref[1] = jnp.concatenate([hs, pad], axis=1)


def _node2_call(x, agg, *ws):
    n = x.shape[0]
    return pl.pallas_call(
        _node2_body,
        out_shape=jax.ShapeDtypeStruct((2, n, 128), _f32),
    )(x, agg, *ws)


def _edge2_body(gd_ref, gs_ref, e1_ref, ea_ref, we, w2, b2, w3, b3, w4, b4,
                v1, c1, v2, c2, v3, c3, v4, c4, o_ref):
    h = we.shape[1]
    e1 = e1_ref[...]
    ea = ea_ref[...]
    t = _relu(gd_ref[:, :h] + gs_ref[:, :h] + _dot(e1, we[...]))
    t = _relu(_dot(t, w2[...]) + b2[...])
    t = _relu(_dot(t, w3[...]) + b3[...])
    e2 = _dot(t, w4[...]) + b4[...]
    z0 = jnp.concatenate([ea, e1, e2], axis=1)
    z = _relu(_dot(z0, v1[...]) + c1[...])
    z = _relu(_dot(z, v2[...]) + c2[...])
    z = _relu(_dot(z, v3[...]) + c3[...])
    logits = jnp.sum(z * v4[...], axis=1, keepdims=True) + c4[...]
    o_ref[...] = 1.0 / (1.0 + jnp.exp(-logits))


def _edge2_call(g, e1h, ea, weights, blk, eh, ea_off):
    nb = eh // blk
    de = ea.shape[1]
    full = lambda a: pl.BlockSpec(a.shape, lambda i: (0,) * a.ndim)
    return pl.pallas_call(
        _edge2_body,
        grid=(nb,),
        in_specs=[
            pl.BlockSpec((blk, 128), lambda i: (i, 0)),
            pl.BlockSpec((blk, 128), lambda i: (i + nb, 0)),
            pl.BlockSpec((blk, de), lambda i: (i, 0)),
            pl.BlockSpec((blk, de), lambda i: (i + ea_off, 0)),
        ] + [full(w) for w in weights],
        out_specs=pl.BlockSpec((blk, 1), lambda i: (i, 0)),
        out_shape=jax.ShapeDtypeStruct((eh, 1), _f32),
    )(g, g, e1h, ea, *weights)


_GW = 128


def _sc_gather(table, idx):
    m = idx.shape[0]
    d = table.shape[1]
    mesh = plsc.VectorSubcoreMesh(core_axis_name="c", subcore_axis_name="s")

    @functools.partial(
        pl.kernel,
        out_type=jax.ShapeDtypeStruct((m, d), _f32),
        mesh=mesh,
    )
    def k(tab_hbm, i_hbm, o_hbm):
        def body(i_vmem, o_vmem):
            pltpu.sync_copy(tab_hbm.at[i_vmem.at[0]], o_vmem)

        pltpu.emit_pipeline(
            body,
            grid=(m // _GW,),
            in_specs=[pl.BlockSpec((1, _GW), lambda i: (0, i))],
            out_specs=[pl.BlockSpec((_GW, d), lambda i: (i, 0))],
            core_axis_name=("c", "s"),
            dimension_semantics=(pltpu.PARALLEL,),
        )(i_hbm, o_hbm)

    return k(table, idx.reshape(1, m))


def _sc_scatter_add(e_a, e_b, d_a, d_b, n_nodes):
    ne, d = e_a.shape
    half = n_nodes // 2
    n_rows = ne // 128
    rows_per_tile = n_rows // 16
    n_ch = rows_per_tile // 4
    mesh = plsc.VectorSubcoreMesh(core_axis_name="c", subcore_axis_name="s")

    @functools.partial(
        pl.kernel,
        out_type=jax.ShapeDtypeStruct((2, 16, half, d), _f32),
        mesh=mesh,
        scratch_types=[
            pltpu.VMEM((512, d), _f32),
            pltpu.VMEM((4, 128), jnp.int32),
            pltpu.VMEM((4, 128), jnp.int32),
            pltpu.VMEM_SHARED((half + 8, d), _f32),
        ],
    )
    def k(ea_hbm, eb_hbm, da_hbm, db_hbm, z_hbm, o_hbm, ebuf, ibuf, lbuf, acc):
        cid = lax.axis_index("c")
        sid = lax.axis_index("s")
        grp = sid // 8
        pos = cid * 8 + sid % 8
        lo = grp * half

        pltpu.sync_copy(z_hbm, acc)

        def work(e_hbm, d_hbm):
            base = sid * rows_per_tile

            @pl.loop(0, n_ch)
            def _(c):
                row0 = base + c * 4
                pltpu.sync_copy(e_hbm.at[pl.ds(row0 * 128, 512)], ebuf)
                pltpu.sync_copy(d_hbm.at[pl.ds(row0, 4)], ibuf)

                for r in range(4):
                    for v in range(8):
                        idx = ibuf[r, pl.ds(v * 16, 16)]
                        loc = idx - lo
                        ok = (loc >= 0) & (loc < half)
                        lbuf[r, pl.ds(v * 16, 16)] = jnp.where(
                            ok, loc, jnp.full((16,), half, jnp.int32)
                        )

                for r in range(4):
                    pltpu.sync_copy(
                        ebuf.at[pl.ds(r * 128, 128)],
                        acc.at[lbuf.at[r]],
                        add=True,
                    )

        @pl.when(cid == 0)
        def _():
            work(ea_hbm, da_hbm)

        @pl.when(cid == 1)
        def _():
            work(eb_hbm, db_hbm)

        pltpu.sync_copy(acc.at[pl.ds(0, half)], o_hbm.at[grp, pos])

    z = jnp.zeros((half + 8, d), _f32)
    return k(e_a, e_b, d_a, d_b, z)


_EDGE_BLK = 8000


def kernel(x, edge_index, edge_attr, params):
    n, dn = x.shape
    e_cnt = edge_index.shape[1]
    de = edge_attr.shape[1]
    r1, o1, r2, w = params["r1"], params["o1"], params["r2"], params["w"]

    src = edge_index[0]
    dst = edge_index[1]
    eh = e_cnt // 2
    e_pad_h = 16 * 512 * -(-eh // (16 * 512))
    pad_i = jnp.full((e_pad_h - eh,), n, jnp.int32)
    halves = []
    for lo in (0, eh):
        d_h = lax.dynamic_slice_in_dim(dst, lo, eh)
        s_h = lax.dynamic_slice_in_dim(src, lo, eh)
        idx_h = jnp.concatenate([d_h, s_h + n])
        d2_h = jnp.concatenate([d_h, pad_i]).reshape(-1, 128)
        halves.append((idx_h, d2_h))

    row = lambda v: v.reshape(1, -1)

    w1, b1 = r1[0]
    t1 = _proj1_call(x, w1[:dn], w1[dn:2 * dn], row(b1)).reshape(2 * n, 128)
    ws1 = [
        w1[2 * dn:],
        r1[1][0], row(r1[1][1]), r1[2][0], row(r1[2][1]),
        r1[3][0], row(r1[3][1]),
    ]
    e1h = []
    for hx, (idx_h, _) in enumerate(halves):
        g_h = _sc_gather(t1, idx_h)
        e1h.append(_edge1_call(g_h, edge_attr, ws1, _EDGE_BLK, eh, e_pad_h,
                               hx * (eh // _EDGE_BLK)))

    agg = _reduce_call(
        _sc_scatter_add(e1h[0], e1h[1], halves[0][1], halves[1][1], n), n)

    wo1, bo1 = o1[0]
    w21, b21 = r2[0]
    t2 = _node2_call(
        x, agg, wo1[:dn], wo1[dn:], row(bo1),
        o1[1][0], row(o1[1][1]), o1[2][0], row(o1[2][1]),
        o1[3][0], row(o1[3][1]),
        w21[:dn], w21[dn:2 * dn], row(b21),
    ).reshape(2 * n, 128)

    wv1, cv1 = w[0]
    ws2 = [
        w21[2 * dn:],
        r2[1][0], row(r2[1][1]), r2[2][0], row(r2[2][1]),
        r2[3][0], row(r2[3][1]),
        wv1, row(cv1),
        w[1][0], row(w[1][1]), w[2][0], row(w[2][1]),
        row(w[3][0].reshape(-1)), row(w[3][1]),
    ]
    ew = []
    for hx, (idx_h, _) in enumerate(halves):
        g_h = _sc_gather(t2, idx_h)
        ew.append(_edge2_call(g_h, e1h[hx], edge_attr, ws2, _EDGE_BLK, eh,
                              hx * (eh // _EDGE_BLK)))
    return jnp.concatenate(ew, axis=0)
